# Initial kernel scaffold; baseline (speedup 1.0000x reference)
import jax
import jax.numpy as jnp
from jax.experimental import pallas as pl


def kernel(u, neighbor_idx, target_idx, entity_table, user_table, aW, ab, hW, hb):
    raise NotImplementedError("write your pallas kernel here")


# speedup vs baseline: 15.3282x; 15.3282x over previous
import functools

import jax
import jax.numpy as jnp
from jax import lax
from jax.experimental import pallas as pl
from jax.experimental.pallas import tpu as pltpu
from jax.experimental.pallas import tpu_sc as plsc

NC, NS, L = 2, 16, 16
NW = NC * NS


def _scale_tables(entity_table, user_table):
    n, k = entity_table.shape
    blk = 4096
    grid = (n + 2 * blk - 1) // (2 * blk)
    n2 = grid * 2 * blk
    assert user_table.shape == (n, k)

    def body(ea, eb, ua, ub, oe_ref, ou_ref):
        for a, b, dst in ((ea, eb, oe_ref), (ua, ub, ou_ref)):
            ys = []
            for src in (a, b):
                x = src[...]
                ss = jnp.sum(x * x, axis=0, keepdims=True)
                scale = jnp.minimum(1.0, 1.0 / (jnp.sqrt(ss) + 1e-7))
                ys.append((x * scale).T)
            dst[...] = jnp.concatenate(ys, axis=1)

    last_blk = (n - 1) // blk
    spec_a = pl.BlockSpec((k, blk), lambda i: (0, 2 * i))
    spec_b = pl.BlockSpec((k, blk),
                          lambda i: (0, jnp.minimum(2 * i + 1, last_blk)))
    out_spec = pl.BlockSpec((blk, 2 * k), lambda i: (i, 0))
    fe, fu = pl.pallas_call(
        body,
        grid=(grid,),
        in_specs=[spec_a, spec_b, spec_a, spec_b],
        out_specs=[out_spec, out_spec],
        out_shape=[jax.ShapeDtypeStruct((n2 // 2, 2 * k), jnp.float32)] * 2,
    )(entity_table.T, entity_table.T, user_table.T, user_table.T)
    return fe.reshape(n2, k), fu.reshape(n2, k)


def _remap_idx(e):
    return (e & -8192) | ((e & 4095) << 1) | ((e >> 12) & 1)


def _sc_forward(nbr_flat, tgt_idx, u_idx, es, us):
    B = tgt_idx.shape[0]
    NN = 32
    K = es.shape[1]
    CHUNK = 16
    b_per_w = B // NW
    n_chunks = b_per_w // CHUNK
    GSPLIT = CHUNK * NN // 128

    mesh = plsc.VectorSubcoreMesh(core_axis_name="c", subcore_axis_name="s")
    ncol = K // L

    buf = lambda: [
        pltpu.VMEM((CHUNK * NN, K), jnp.float32),
        pltpu.VMEM((CHUNK, K), jnp.float32),
        pltpu.VMEM((CHUNK, K), jnp.float32),
        pltpu.SemaphoreType.DMA,
    ]

    @functools.partial(
        pl.kernel,
        out_type=jax.ShapeDtypeStruct((B,), jnp.float32),
        mesh=mesh,
        compiler_params=pltpu.CompilerParams(needs_layout_passes=False,
                                             use_tc_tiling_on_sc=False),
        scratch_types=buf() + buf() + [
            pltpu.VMEM((b_per_w * NN,), jnp.int32),
            pltpu.VMEM((b_per_w,), jnp.int32),
            pltpu.VMEM((b_per_w,), jnp.int32),
            pltpu.VMEM((CHUNK * L,), jnp.float32),
            pltpu.VMEM((b_per_w,), jnp.float32),
        ],
    )
    def k(nbr_hbm, tgt_hbm, u_hbm, es_hbm, us_hbm, out_hbm, *scr):
        bufs = (scr[0:4], scr[4:8])
        nidx_all, tidx_all, uidx_all, psum_v, out_v = scr[8:13]
        wid = lax.axis_index("s") * NC + lax.axis_index("c")
        lanes = lax.iota(jnp.int32, L)

        def copies(ci, ph):
            rows_v, trow_v, urow_v, sem = bufs[ph]
            cps = [
                pltpu.make_async_copy(
                    es_hbm.at[nidx_all.at[pl.ds(ci * CHUNK * NN + j * 128, 128)]],
                    rows_v.at[pl.ds(j * 128, 128)], sem)
                for j in range(GSPLIT)
            ]
            cps.append(pltpu.make_async_copy(
                es_hbm.at[tidx_all.at[pl.ds(ci * CHUNK, CHUNK)]], trow_v, sem))
            cps.append(pltpu.make_async_copy(
                us_hbm.at[uidx_all.at[pl.ds(ci * CHUNK, CHUNK)]], urow_v, sem))
            return cps

        def fetch(ci, ph):
            for cp in copies(ci, ph):
                cp.start()

        def compute(ci, ph):
            rows_v, trow_v, urow_v, _ = bufs[ph]
            base = wid * b_per_w + ci * CHUNK
            cps = copies(ci, ph)
            for cp in cps:
                cp.wait()

            def elem_body(b):
                zero = jnp.zeros((L,), jnp.float32)
                s = [zero] * ncol
                q = [zero] * ncol
                for n in range(NN):
                    row = b * NN + n
                    for c in range(ncol):
                        x = rows_v[row, pl.ds(c * L, L)]
                        s[c] = s[c] + x
                        q[c] = q[c] + x * x
                p = zero
                for c in range(ncol):
                    t = trow_v[b, pl.ds(c * L, L)]
                    uu = urow_v[b, pl.ds(c * L, L)]
                    p = p + uu * (s[c] * s[c] - q[c] + t)
                psum_v[pl.ds(b * L, L)] = p

            @pl.loop(0, CHUNK, step=2)
            def _(b):
                elem_body(b)
                elem_body(b + 1)
            uv = jnp.zeros((L,), jnp.float32)
            for c in range(L):
                uv = uv + plsc.load_gather(psum_v, [lanes * L + c])
            out_v[pl.ds(ci * CHUNK, CHUNK)] = 1.0 / (1.0 + jnp.exp(-uv))

        wbase = wid * b_per_w
        pltpu.sync_copy(nbr_hbm.at[pl.ds(wbase * NN, b_per_w * NN)], nidx_all)
        pltpu.sync_copy(tgt_hbm.at[pl.ds(wbase, b_per_w)], tidx_all)
        pltpu.sync_copy(u_hbm.at[pl.ds(wbase, b_per_w)], uidx_all)
        fetch(0, 0)

        @pl.loop(0, n_chunks // 2)
        def pair_body(cp):
            for ph in (0, 1):
                ci = cp * 2 + ph
                nxt = ci + 1

                @pl.when(nxt < n_chunks)
                def _():
                    fetch(nxt, 1 - ph)

                compute(ci, ph)

        pltpu.sync_copy(out_v, out_hbm.at[pl.ds(wbase, b_per_w)])

    return k(nbr_flat, tgt_idx, u_idx, es, us)


def kernel(u, neighbor_idx, target_idx, entity_table, user_table, aW, ab, hW, hb):
    del aW, ab, hW, hb
    es, us = _scale_tables(entity_table, user_table)
    nbr_flat = _remap_idx(neighbor_idx.astype(jnp.int32)).reshape(-1)
    return _sc_forward(nbr_flat, _remap_idx(target_idx.astype(jnp.int32)),
                       _remap_idx(u.astype(jnp.int32)), es, us)
